# baseline (speedup 1.0000x reference)
import jax
import jax.numpy as jnp
from jax import lax
from jax.experimental import pallas as pl
from jax.experimental.pallas import tpu as pltpu
from jax.experimental.pallas import tpu_sc as plsc

_N_DATA = 1000000
_N_CLASSES = 100
_BATCH = 16384

_NC, _NS = 2, 16
_NW = _NC * _NS
_BPW = _BATCH // _NW
_CHUNK = 128
_NCHUNK = _BPW // _CHUNK
_LAG = 96

_SC_MESH = plsc.VectorSubcoreMesh(core_axis_name="c", subcore_axis_name="s")
_SC_PARAMS = pltpu.CompilerParams(needs_layout_passes=False)


def _gather_body(conf_hbm, gidx_hbm, r8_hbm, out_hbm,
                 gidx_v, r8_v, rows_v, sem):
    conf3 = conf_hbm.reshape(_N_DATA // 8, 8, _N_CLASSES)
    wid = lax.axis_index("s") * _NC + lax.axis_index("c")
    base = wid * _BPW
    pltpu.sync_copy(gidx_hbm.at[pl.ds(wid * _NCHUNK, _NCHUNK)], gidx_v)
    pltpu.sync_copy(r8_hbm.at[pl.ds(wid * _NCHUNK, _NCHUNK)], r8_v)
    lanes = lax.iota(jnp.int32, 16)

    def vec_body(q, _):
        gv = gidx_v[q // 8, pl.ds((q % 8) * 16, 16)]
        rv = r8_v[q // 8, pl.ds((q % 8) * 16, 16)]
        for l in range(16):
            p = q * 16 + l
            sel = lanes == l
            g = jnp.max(jnp.where(sel, gv, 0))
            r8 = jnp.max(jnp.where(sel, rv, 0))
            pltpu.async_copy(
                conf3.at[g, r8], rows_v.at[p, pl.ds(0, _N_CLASSES)], sem)
        return 0

    lax.fori_loop(0, _BPW // 16, vec_body, 0)

    def drain_body(p, _):
        pltpu.make_async_copy(
            conf3.at[0, 0], rows_v.at[0, pl.ds(0, _N_CLASSES)], sem).wait()
        return 0
    lax.fori_loop(0, _BPW, drain_body, 0)

    pltpu.sync_copy(rows_v, out_hbm.at[pl.ds(base, _BPW)])


def _sc_gather(confidence, gidx2d, r82d):
    return pl.kernel(
        _gather_body,
        out_type=jax.ShapeDtypeStruct((_BATCH, 128), jnp.float32),
        mesh=_SC_MESH,
        scratch_types=[
            pltpu.VMEM((_NCHUNK, _CHUNK), jnp.int32),
            pltpu.VMEM((_NCHUNK, _CHUNK), jnp.int32),
            pltpu.VMEM((_BPW, 128), jnp.float32),
            pltpu.SemaphoreType.DMA,
        ],
        compiler_params=_SC_PARAMS,
    )(confidence, gidx2d, r82d)


_ROWS_PER_BLOCK = 2048
_N_BLOCKS = _BATCH // _ROWS_PER_BLOCK


def _compute_body(o_ref, t_ref, nt_ref, loss_ref):
    pid = pl.program_id(0)

    x = o_ref[...]
    t = t_ref[:, :_N_CLASSES]
    m = jnp.max(x, axis=1, keepdims=True)
    e = jnp.exp(x - m)
    s = jnp.sum(e, axis=1, keepdims=True)
    p = e / s
    logp = (x - m) - jnp.log(s)
    block_loss = jnp.sum(t * logp)

    r = jnp.where(t > 0, p, jnp.zeros_like(p))
    nt = r / jnp.sum(r, axis=1, keepdims=True)
    nt_ref[...] = nt

    @pl.when(pid == 0)
    def _():
        loss_ref[0, 0] = 0.0

    loss_ref[0, 0] += -block_loss / _BATCH


def _compute_tc(output1, target128):
    nt, loss = pl.pallas_call(
        _compute_body,
        grid=(_N_BLOCKS,),
        in_specs=[
            pl.BlockSpec((_ROWS_PER_BLOCK, _N_CLASSES), lambda i: (i, 0)),
            pl.BlockSpec((_ROWS_PER_BLOCK, 128), lambda i: (i, 0)),
        ],
        out_specs=[
            pl.BlockSpec((_ROWS_PER_BLOCK, _N_CLASSES), lambda i: (i, 0)),
            pl.BlockSpec(memory_space=pltpu.SMEM, block_shape=(1, 1),
                         index_map=lambda i: (0, 0)),
        ],
        out_shape=[
            jax.ShapeDtypeStruct((_BATCH, _N_CLASSES), jnp.float32),
            jax.ShapeDtypeStruct((1, 1), jnp.float32),
        ],
    )(output1, target128)
    return loss[0, 0], nt


_COPY_ROWS = 20000
_COPY_BLOCKS = _N_DATA // _COPY_ROWS


def _copy_body(src_ref, dst_ref):
    dst_ref[...] = src_ref[...]


def _tc_copy(confidence):
    return pl.pallas_call(
        _copy_body,
        grid=(_COPY_BLOCKS,),
        in_specs=[pl.BlockSpec((_COPY_ROWS, _N_CLASSES), lambda i: (i, 0))],
        out_specs=pl.BlockSpec((_COPY_ROWS, _N_CLASSES), lambda i: (i, 0)),
        out_shape=jax.ShapeDtypeStruct((_N_DATA, _N_CLASSES), jnp.float32),
    )(confidence)


_NGRP = _N_DATA // 8
_GPW = _NGRP // _NW
_RPW = _GPW * 8
_XTRA_ROW0 = _RPW * _NW
_WSZ = _RPW + (_N_DATA - _XTRA_ROW0) + 16
_NVEC = _BATCH // 16


def _plan_body(idx_hbm, loc_hbm, pos_hbm, cnt_hbm,
               idx_v, w_v, loc_v, pos_v, cnt_v):
    wid = lax.axis_index("s") * _NC + lax.axis_index("c")
    r_lo = wid * _RPW
    lanes = lax.iota(jnp.int32, 16)

    pltpu.sync_copy(idx_hbm, idx_v)

    xtra_thr = jnp.where(wid == 0, _XTRA_ROW0, _N_DATA)

    def owned_loc(iv):
        m_main = (iv >= r_lo) & (iv < r_lo + _RPW)
        m_x = iv >= xtra_thr
        m = m_main | m_x
        loc = jnp.where(m_main, iv - r_lo,
                        jnp.where(m_x, iv - _XTRA_ROW0 + _RPW, 0))
        return m, loc

    def scan_body(q, _):
        iv = idx_v[q // 8, pl.ds((q % 8) * 16, 16)]
        m, loc = owned_loc(iv)
        pos = q * 16 + lanes
        for l in range(16):
            plsc.store_scatter(w_v, [loc], pos, mask=m & (lanes == l))
        return 0
    lax.fori_loop(0, _NVEC, scan_body, 0)

    def live_body(q, off):
        iv = idx_v[q // 8, pl.ds((q % 8) * 16, 16)]
        m, loc = owned_loc(iv)
        pos = q * 16 + lanes
        got = plsc.load_gather(w_v, [loc], mask=m)
        live = m & (got == pos)
        cum = plsc.cumsum(live.astype(jnp.int32))
        tgt = off + cum - 1
        tr = lax.shift_right_logical(tgt, 7)
        tc = tgt & (_CHUNK - 1)
        plsc.store_scatter(loc_v, [tr, tc], loc, mask=live)
        plsc.store_scatter(pos_v, [tr, tc], pos, mask=live)
        return off + jnp.sum(live.astype(jnp.int32))
    n_live = lax.fori_loop(0, _NVEC, live_body, 0)

    cnt_v[pl.ds(0, 16)] = jnp.where(lanes == 0, n_live, 0)
    pltpu.sync_copy(loc_v, loc_hbm.at[wid])
    pltpu.sync_copy(pos_v, pos_hbm.at[wid])
    pltpu.sync_copy(cnt_v, cnt_hbm.at[wid])


def _sc_plan(idx2d):
    return pl.kernel(
        _plan_body,
        out_type=[
            jax.ShapeDtypeStruct((_NW, _BATCH // _CHUNK, _CHUNK),
                                 jnp.int32),
            jax.ShapeDtypeStruct((_NW, _BATCH // _CHUNK, _CHUNK),
                                 jnp.int32),
            jax.ShapeDtypeStruct((_NW, 16), jnp.int32),
        ],
        mesh=_SC_MESH,
        scratch_types=[
            pltpu.VMEM((_BATCH // _CHUNK, _CHUNK), jnp.int32),
            pltpu.VMEM((_WSZ,), jnp.int32),
            pltpu.VMEM((_BATCH // _CHUNK, _CHUNK), jnp.int32),
            pltpu.VMEM((_BATCH // _CHUNK, _CHUNK), jnp.int32),
            pltpu.VMEM((16,), jnp.int32),
        ],
        compiler_params=_SC_PARAMS,
    )(idx2d)


def _write_rows(table_ref, nt_ref, loc_ref, pos_ref, cnt_ref,
                loc_v, pos_v, cnt_v, rsem):
    tab3 = table_ref.reshape(_NGRP, 8, _N_CLASSES)
    nt3 = nt_ref.reshape(_BATCH // 8, 8, _N_CLASSES)
    wid = lax.axis_index("s") * _NC + lax.axis_index("c")
    r_lo = wid * _RPW
    lanes = lax.iota(jnp.int32, 16)

    pltpu.sync_copy(loc_ref.at[wid], loc_v)
    pltpu.sync_copy(pos_ref.at[wid], pos_v)
    pltpu.sync_copy(cnt_ref.at[wid], cnt_v)
    n_live = jnp.max(jnp.where(lanes == 0, cnt_v[pl.ds(0, 16)], 0))

    def row_body(e, _):
        lv = loc_v[e // _CHUNK, pl.ds(((e // 16) % 8) * 16, 16)]
        pv = pos_v[e // _CHUNK, pl.ds(((e // 16) % 8) * 16, 16)]
        sel = lanes == (e % 16)
        loc = jnp.max(jnp.where(sel, lv, 0))
        pos = jnp.max(jnp.where(sel, pv, 0))
        row = jnp.where(loc < _RPW, loc + r_lo, loc - _RPW + _XTRA_ROW0)
        pltpu.async_copy(
            nt3.at[lax.shift_right_logical(pos, 3), pos & 7],
            tab3.at[lax.shift_right_logical(row, 3), row & 7], rsem)
        return 0
    lax.fori_loop(0, n_live, row_body, 0)

    def drain8(e, _):
        pltpu.make_async_copy(nt3.at[0], tab3.at[0], rsem).wait()
        return 0
    lax.fori_loop(0, lax.shift_right_logical(n_live, 3), drain8, 0)

    def drain1(e, _):
        pltpu.make_async_copy(nt3.at[0, 0], tab3.at[0, 0], rsem).wait()
        return 0
    lax.fori_loop(0, n_live & 7, drain1, 0)


def _sc_scatter(table, new_target, loc, pos, cnt):
    def stateful(refs):
        table_ref, nt_ref, loc_ref, pos_ref, cnt_ref = refs

        @pl.core_map(_SC_MESH, compiler_params=_SC_PARAMS,
                     scratch_shapes=[
                         pltpu.VMEM((_BATCH // _CHUNK, _CHUNK), jnp.int32),
                         pltpu.VMEM((_BATCH // _CHUNK, _CHUNK), jnp.int32),
                         pltpu.VMEM((16,), jnp.int32),
                         pltpu.SemaphoreType.DMA,
                     ])
        def _(loc_v, pos_v, cnt_v, rsem):
            _write_rows(table_ref, nt_ref, loc_ref, pos_ref, cnt_ref,
                        loc_v, pos_v, cnt_v, rsem)

    outs = pl.run_state(stateful)((table, new_target, loc, pos, cnt))
    return outs[0]


def kernel(output1, index, confidence):
    gidx2d = (index // 8).reshape(_BATCH // _CHUNK, _CHUNK)
    r82d = (index & 7).reshape(_BATCH // _CHUNK, _CHUNK)
    idx2d = index.reshape(_BATCH // _CHUNK, _CHUNK)
    loc, pos, cnt = _sc_plan(idx2d)
    target128 = _sc_gather(confidence, gidx2d, r82d)
    loss, new_target = _compute_tc(output1, target128)
    table = _tc_copy(confidence)
    new_confidence = _sc_scatter(table, new_target, loc, pos, cnt)
    return loss, new_confidence


# speedup vs baseline: 1.0003x; 1.0003x over previous
import jax
import jax.numpy as jnp
from jax import lax
from jax.experimental import pallas as pl
from jax.experimental.pallas import tpu as pltpu
from jax.experimental.pallas import tpu_sc as plsc

_N_DATA = 1000000
_N_CLASSES = 100
_BATCH = 16384

_NC, _NS = 2, 16
_NW = _NC * _NS
_BPW = _BATCH // _NW
_CHUNK = 128
_NCHUNK = _BPW // _CHUNK
_LAG = 96

_SC_MESH = plsc.VectorSubcoreMesh(core_axis_name="c", subcore_axis_name="s")
_SC_PARAMS = pltpu.CompilerParams(needs_layout_passes=False,
                                  use_tc_tiling_on_sc=True)


def _gather_body(conf_hbm, gidx_hbm, r8_hbm, out_hbm,
                 gidx_v, r8_v, rows_v, sem):
    conf3 = conf_hbm.reshape(_N_DATA // 8, 8, _N_CLASSES)
    wid = lax.axis_index("s") * _NC + lax.axis_index("c")
    base = wid * _BPW
    pltpu.sync_copy(gidx_hbm.at[pl.ds(wid * _NCHUNK, _NCHUNK)], gidx_v)
    pltpu.sync_copy(r8_hbm.at[pl.ds(wid * _NCHUNK, _NCHUNK)], r8_v)
    lanes = lax.iota(jnp.int32, 16)

    def vec_body(q, _):
        gv = gidx_v[q // 8, pl.ds((q % 8) * 16, 16)]
        rv = r8_v[q // 8, pl.ds((q % 8) * 16, 16)]
        for l in range(16):
            p = q * 16 + l
            sel = lanes == l
            g = jnp.max(jnp.where(sel, gv, 0))
            r8 = jnp.max(jnp.where(sel, rv, 0))
            pltpu.async_copy(
                conf3.at[g, r8], rows_v.at[p, pl.ds(0, _N_CLASSES)], sem)
        return 0

    lax.fori_loop(0, _BPW // 16, vec_body, 0)

    def drain_body(p, _):
        pltpu.make_async_copy(
            conf3.at[0, 0], rows_v.at[0, pl.ds(0, _N_CLASSES)], sem).wait()
        return 0
    lax.fori_loop(0, _BPW, drain_body, 0)

    pltpu.sync_copy(rows_v, out_hbm.at[pl.ds(base, _BPW)])


def _sc_gather(confidence, gidx2d, r82d):
    return pl.kernel(
        _gather_body,
        out_type=jax.ShapeDtypeStruct((_BATCH, 128), jnp.float32),
        mesh=_SC_MESH,
        scratch_types=[
            pltpu.VMEM((_NCHUNK, _CHUNK), jnp.int32),
            pltpu.VMEM((_NCHUNK, _CHUNK), jnp.int32),
            pltpu.VMEM((_BPW, 128), jnp.float32),
            pltpu.SemaphoreType.DMA,
        ],
        compiler_params=_SC_PARAMS,
    )(confidence, gidx2d, r82d)


_ROWS_PER_BLOCK = 2048
_N_BLOCKS = _BATCH // _ROWS_PER_BLOCK


def _compute_body(o_ref, t_ref, nt_ref, loss_ref):
    pid = pl.program_id(0)

    x = o_ref[...]
    t = t_ref[:, :_N_CLASSES]
    m = jnp.max(x, axis=1, keepdims=True)
    e = jnp.exp(x - m)
    s = jnp.sum(e, axis=1, keepdims=True)
    p = e / s
    logp = (x - m) - jnp.log(s)
    block_loss = jnp.sum(t * logp)

    r = jnp.where(t > 0, p, jnp.zeros_like(p))
    nt = r / jnp.sum(r, axis=1, keepdims=True)
    nt_ref[...] = nt

    @pl.when(pid == 0)
    def _():
        loss_ref[0, 0] = 0.0

    loss_ref[0, 0] += -block_loss / _BATCH


def _compute_tc(output1, target128):
    nt, loss = pl.pallas_call(
        _compute_body,
        grid=(_N_BLOCKS,),
        in_specs=[
            pl.BlockSpec((_ROWS_PER_BLOCK, _N_CLASSES), lambda i: (i, 0)),
            pl.BlockSpec((_ROWS_PER_BLOCK, 128), lambda i: (i, 0)),
        ],
        out_specs=[
            pl.BlockSpec((_ROWS_PER_BLOCK, _N_CLASSES), lambda i: (i, 0)),
            pl.BlockSpec(memory_space=pltpu.SMEM, block_shape=(1, 1),
                         index_map=lambda i: (0, 0)),
        ],
        out_shape=[
            jax.ShapeDtypeStruct((_BATCH, _N_CLASSES), jnp.float32),
            jax.ShapeDtypeStruct((1, 1), jnp.float32),
        ],
    )(output1, target128)
    return loss[0, 0], nt


_COPY_ROWS = 20000
_COPY_BLOCKS = _N_DATA // _COPY_ROWS


def _copy_body(src_ref, dst_ref):
    dst_ref[...] = src_ref[...]


def _tc_copy(confidence):
    return pl.pallas_call(
        _copy_body,
        grid=(_COPY_BLOCKS,),
        in_specs=[pl.BlockSpec((_COPY_ROWS, _N_CLASSES), lambda i: (i, 0))],
        out_specs=pl.BlockSpec((_COPY_ROWS, _N_CLASSES), lambda i: (i, 0)),
        out_shape=jax.ShapeDtypeStruct((_N_DATA, _N_CLASSES), jnp.float32),
    )(confidence)


_NGRP = _N_DATA // 8
_GPW = _NGRP // _NW
_RPW = _GPW * 8
_XTRA_ROW0 = _RPW * _NW
_WSZ = _RPW + (_N_DATA - _XTRA_ROW0) + 16
_NVEC = _BATCH // 16


def _plan_body(idx_hbm, loc_hbm, pos_hbm, cnt_hbm,
               idx_v, w_v, loc_v, pos_v, cnt_v):
    wid = lax.axis_index("s") * _NC + lax.axis_index("c")
    r_lo = wid * _RPW
    lanes = lax.iota(jnp.int32, 16)

    pltpu.sync_copy(idx_hbm, idx_v)

    xtra_thr = jnp.where(wid == 0, _XTRA_ROW0, _N_DATA)

    def owned_loc(iv):
        m_main = (iv >= r_lo) & (iv < r_lo + _RPW)
        m_x = iv >= xtra_thr
        m = m_main | m_x
        loc = jnp.where(m_main, iv - r_lo,
                        jnp.where(m_x, iv - _XTRA_ROW0 + _RPW, 0))
        return m, loc

    def scan_body(q, _):
        iv = idx_v[q // 8, pl.ds((q % 8) * 16, 16)]
        m, loc = owned_loc(iv)
        pos = q * 16 + lanes
        for l in range(16):
            plsc.store_scatter(w_v, [loc], pos, mask=m & (lanes == l))
        return 0
    lax.fori_loop(0, _NVEC, scan_body, 0)

    def live_body(q, off):
        iv = idx_v[q // 8, pl.ds((q % 8) * 16, 16)]
        m, loc = owned_loc(iv)
        pos = q * 16 + lanes
        got = plsc.load_gather(w_v, [loc], mask=m)
        live = m & (got == pos)
        cum = plsc.cumsum(live.astype(jnp.int32))
        tgt = off + cum - 1
        tr = lax.shift_right_logical(tgt, 7)
        tc = tgt & (_CHUNK - 1)
        plsc.store_scatter(loc_v, [tr, tc], loc, mask=live)
        plsc.store_scatter(pos_v, [tr, tc], pos, mask=live)
        return off + jnp.sum(live.astype(jnp.int32))
    n_live = lax.fori_loop(0, _NVEC, live_body, 0)

    cnt_v[pl.ds(0, 16)] = jnp.where(lanes == 0, n_live, 0)
    pltpu.sync_copy(loc_v, loc_hbm.at[wid])
    pltpu.sync_copy(pos_v, pos_hbm.at[wid])
    pltpu.sync_copy(cnt_v, cnt_hbm.at[wid])


def _sc_plan(idx2d):
    return pl.kernel(
        _plan_body,
        out_type=[
            jax.ShapeDtypeStruct((_NW, _BATCH // _CHUNK, _CHUNK),
                                 jnp.int32),
            jax.ShapeDtypeStruct((_NW, _BATCH // _CHUNK, _CHUNK),
                                 jnp.int32),
            jax.ShapeDtypeStruct((_NW, 16), jnp.int32),
        ],
        mesh=_SC_MESH,
        scratch_types=[
            pltpu.VMEM((_BATCH // _CHUNK, _CHUNK), jnp.int32),
            pltpu.VMEM((_WSZ,), jnp.int32),
            pltpu.VMEM((_BATCH // _CHUNK, _CHUNK), jnp.int32),
            pltpu.VMEM((_BATCH // _CHUNK, _CHUNK), jnp.int32),
            pltpu.VMEM((16,), jnp.int32),
        ],
        compiler_params=_SC_PARAMS,
    )(idx2d)


def _write_rows(table_ref, nt_ref, loc_ref, pos_ref, cnt_ref,
                loc_v, pos_v, cnt_v, rsem):
    tab3 = table_ref.reshape(_NGRP, 8, _N_CLASSES)
    nt3 = nt_ref.reshape(_BATCH // 8, 8, _N_CLASSES)
    wid = lax.axis_index("s") * _NC + lax.axis_index("c")
    r_lo = wid * _RPW
    lanes = lax.iota(jnp.int32, 16)

    pltpu.sync_copy(loc_ref.at[wid], loc_v)
    pltpu.sync_copy(pos_ref.at[wid], pos_v)
    pltpu.sync_copy(cnt_ref.at[wid], cnt_v)
    n_live = jnp.max(jnp.where(lanes == 0, cnt_v[pl.ds(0, 16)], 0))

    def row_body(e, _):
        lv = loc_v[e // _CHUNK, pl.ds(((e // 16) % 8) * 16, 16)]
        pv = pos_v[e // _CHUNK, pl.ds(((e // 16) % 8) * 16, 16)]
        sel = lanes == (e % 16)
        loc = jnp.max(jnp.where(sel, lv, 0))
        pos = jnp.max(jnp.where(sel, pv, 0))
        row = jnp.where(loc < _RPW, loc + r_lo, loc - _RPW + _XTRA_ROW0)
        pltpu.async_copy(
            nt3.at[lax.shift_right_logical(pos, 3), pos & 7],
            tab3.at[lax.shift_right_logical(row, 3), row & 7], rsem)
        return 0
    lax.fori_loop(0, n_live, row_body, 0)

    def drain8(e, _):
        pltpu.make_async_copy(nt3.at[0], tab3.at[0], rsem).wait()
        return 0
    lax.fori_loop(0, lax.shift_right_logical(n_live, 3), drain8, 0)

    def drain1(e, _):
        pltpu.make_async_copy(nt3.at[0, 0], tab3.at[0, 0], rsem).wait()
        return 0
    lax.fori_loop(0, n_live & 7, drain1, 0)


def _sc_scatter(table, new_target, loc, pos, cnt):
    def stateful(refs):
        table_ref, nt_ref, loc_ref, pos_ref, cnt_ref = refs

        @pl.core_map(_SC_MESH, compiler_params=_SC_PARAMS,
                     scratch_shapes=[
                         pltpu.VMEM((_BATCH // _CHUNK, _CHUNK), jnp.int32),
                         pltpu.VMEM((_BATCH // _CHUNK, _CHUNK), jnp.int32),
                         pltpu.VMEM((16,), jnp.int32),
                         pltpu.SemaphoreType.DMA,
                     ])
        def _(loc_v, pos_v, cnt_v, rsem):
            _write_rows(table_ref, nt_ref, loc_ref, pos_ref, cnt_ref,
                        loc_v, pos_v, cnt_v, rsem)

    outs = pl.run_state(stateful)((table, new_target, loc, pos, cnt))
    return outs[0]


def kernel(output1, index, confidence):
    gidx2d = (index // 8).reshape(_BATCH // _CHUNK, _CHUNK)
    r82d = (index & 7).reshape(_BATCH // _CHUNK, _CHUNK)
    idx2d = index.reshape(_BATCH // _CHUNK, _CHUNK)
    loc, pos, cnt = _sc_plan(idx2d)
    target128 = _sc_gather(confidence, gidx2d, r82d)
    loss, new_target = _compute_tc(output1, target128)
    table = _tc_copy(confidence)
    new_confidence = _sc_scatter(table, new_target, loc, pos, cnt)
    return loss, new_confidence


# speedup vs baseline: 1.2459x; 1.2454x over previous
import jax
import jax.numpy as jnp
from jax import lax
from jax.experimental import pallas as pl
from jax.experimental.pallas import tpu as pltpu
from jax.experimental.pallas import tpu_sc as plsc

_N_DATA = 1000000
_N_CLASSES = 100
_BATCH = 16384

_NC, _NS = 2, 16
_NW = _NC * _NS
_BPW = _BATCH // _NW
_CHUNK = 128
_NCHUNK = _BPW // _CHUNK
_LAG = 96

_SC_MESH = plsc.VectorSubcoreMesh(core_axis_name="c", subcore_axis_name="s")
_SC_PARAMS = pltpu.CompilerParams(needs_layout_passes=False,
                                  use_tc_tiling_on_sc=True)


def _gather_body(conf_hbm, gidx_hbm, r8_hbm, out_hbm,
                 gidx_v, r8_v, rows_v, sem):
    conf3 = conf_hbm.reshape(_N_DATA // 8, 8, _N_CLASSES)
    wid = lax.axis_index("s") * _NC + lax.axis_index("c")
    base = wid * _BPW
    pltpu.sync_copy(gidx_hbm.at[pl.ds(wid * _NCHUNK, _NCHUNK)], gidx_v)
    pltpu.sync_copy(r8_hbm.at[pl.ds(wid * _NCHUNK, _NCHUNK)], r8_v)
    lanes = lax.iota(jnp.int32, 16)

    def vec_body(q, _):
        gv = gidx_v[q // 8, pl.ds((q % 8) * 16, 16)]
        rv = r8_v[q // 8, pl.ds((q % 8) * 16, 16)]
        for l in range(16):
            p = q * 16 + l
            sel = lanes == l
            g = jnp.max(jnp.where(sel, gv, 0))
            r8 = jnp.max(jnp.where(sel, rv, 0))
            pltpu.async_copy(
                conf3.at[g, r8], rows_v.at[p, pl.ds(0, _N_CLASSES)], sem)
        return 0

    lax.fori_loop(0, _BPW // 16, vec_body, 0)

    def drain_body(p, _):
        pltpu.make_async_copy(
            conf3.at[0, 0], rows_v.at[0, pl.ds(0, _N_CLASSES)], sem).wait()
        return 0
    lax.fori_loop(0, _BPW, drain_body, 0)

    pltpu.sync_copy(rows_v, out_hbm.at[pl.ds(base, _BPW)])


def _sc_gather(confidence, gidx2d, r82d):
    return pl.kernel(
        _gather_body,
        out_type=jax.ShapeDtypeStruct((_BATCH, 128), jnp.float32),
        mesh=_SC_MESH,
        scratch_types=[
            pltpu.VMEM((_NCHUNK, _CHUNK), jnp.int32),
            pltpu.VMEM((_NCHUNK, _CHUNK), jnp.int32),
            pltpu.VMEM((_BPW, 128), jnp.float32),
            pltpu.SemaphoreType.DMA,
        ],
        compiler_params=_SC_PARAMS,
    )(confidence, gidx2d, r82d)


_ROWS_PER_BLOCK = 2048
_N_BLOCKS = _BATCH // _ROWS_PER_BLOCK


def _compute_body(o_ref, t_ref, nt_ref, loss_ref):
    pid = pl.program_id(0)

    x = o_ref[...]
    t = t_ref[:, :_N_CLASSES]
    m = jnp.max(x, axis=1, keepdims=True)
    e = jnp.exp(x - m)
    s = jnp.sum(e, axis=1, keepdims=True)
    p = e / s
    logp = (x - m) - jnp.log(s)
    block_loss = jnp.sum(t * logp)

    r = jnp.where(t > 0, p, jnp.zeros_like(p))
    nt = r / jnp.sum(r, axis=1, keepdims=True)
    nt_ref[...] = nt

    @pl.when(pid == 0)
    def _():
        loss_ref[0, 0] = 0.0

    loss_ref[0, 0] += -block_loss / _BATCH


def _compute_tc(output1, target128):
    nt, loss = pl.pallas_call(
        _compute_body,
        grid=(_N_BLOCKS,),
        in_specs=[
            pl.BlockSpec((_ROWS_PER_BLOCK, _N_CLASSES), lambda i: (i, 0)),
            pl.BlockSpec((_ROWS_PER_BLOCK, 128), lambda i: (i, 0)),
        ],
        out_specs=[
            pl.BlockSpec((_ROWS_PER_BLOCK, _N_CLASSES), lambda i: (i, 0)),
            pl.BlockSpec(memory_space=pltpu.SMEM, block_shape=(1, 1),
                         index_map=lambda i: (0, 0)),
        ],
        out_shape=[
            jax.ShapeDtypeStruct((_BATCH, _N_CLASSES), jnp.float32),
            jax.ShapeDtypeStruct((1, 1), jnp.float32),
        ],
    )(output1, target128)
    return loss[0, 0], nt


_COPY_ROWS = 20000
_COPY_BLOCKS = _N_DATA // _COPY_ROWS


def _copy_body(src_ref, dst_ref):
    dst_ref[...] = src_ref[...]


def _tc_copy(confidence):
    return pl.pallas_call(
        _copy_body,
        grid=(_COPY_BLOCKS,),
        in_specs=[pl.BlockSpec((_COPY_ROWS, _N_CLASSES), lambda i: (i, 0))],
        out_specs=pl.BlockSpec((_COPY_ROWS, _N_CLASSES), lambda i: (i, 0)),
        out_shape=jax.ShapeDtypeStruct((_N_DATA, _N_CLASSES), jnp.float32),
    )(confidence)


_NGRP = _N_DATA // 8
_GPW = _NGRP // _NW
_RPW = _GPW * 8
_XTRA_ROW0 = _RPW * _NW
_WSZ = _RPW + (_N_DATA - _XTRA_ROW0) + 16
_NVEC = _BATCH // 16


def _plan_body(idx_hbm, loc_hbm, pos_hbm, cnt_hbm,
               idx_v, w_v, loc_v, pos_v, cnt_v):
    wid = lax.axis_index("s") * _NC + lax.axis_index("c")
    r_lo = wid * _RPW
    lanes = lax.iota(jnp.int32, 16)

    pltpu.sync_copy(idx_hbm, idx_v)

    xtra_thr = jnp.where(wid == 0, _XTRA_ROW0, _N_DATA)

    def owned_loc(iv):
        m_main = (iv >= r_lo) & (iv < r_lo + _RPW)
        m_x = iv >= xtra_thr
        m = m_main | m_x
        loc = jnp.where(m_main, iv - r_lo,
                        jnp.where(m_x, iv - _XTRA_ROW0 + _RPW, 0))
        return m, loc

    def scan_body(q, _):
        iv = idx_v[q // 8, pl.ds((q % 8) * 16, 16)]
        m, loc = owned_loc(iv)
        pos = q * 16 + lanes
        for l in range(16):
            plsc.store_scatter(w_v, [loc], pos, mask=m & (lanes == l))
        return 0
    lax.fori_loop(0, _NVEC, scan_body, 0)

    def live_body(q, off):
        iv = idx_v[q // 8, pl.ds((q % 8) * 16, 16)]
        m, loc = owned_loc(iv)
        pos = q * 16 + lanes
        got = plsc.load_gather(w_v, [loc], mask=m)
        live = m & (got == pos)
        cum = plsc.cumsum(live.astype(jnp.int32))
        tgt = off + cum - 1
        tr = lax.shift_right_logical(tgt, 7)
        tc = tgt & (_CHUNK - 1)
        plsc.store_scatter(loc_v, [tr, tc], loc, mask=live)
        plsc.store_scatter(pos_v, [tr, tc], pos, mask=live)
        return off + jnp.sum(live.astype(jnp.int32))
    n_live = lax.fori_loop(0, _NVEC, live_body, 0)

    cnt_v[pl.ds(0, 16)] = jnp.where(lanes == 0, n_live, 0)
    pltpu.sync_copy(loc_v, loc_hbm.at[wid])
    pltpu.sync_copy(pos_v, pos_hbm.at[wid])
    pltpu.sync_copy(cnt_v, cnt_hbm.at[wid])


def _sc_plan(idx2d):
    return pl.kernel(
        _plan_body,
        out_type=[
            jax.ShapeDtypeStruct((_NW, _BATCH // _CHUNK, _CHUNK),
                                 jnp.int32),
            jax.ShapeDtypeStruct((_NW, _BATCH // _CHUNK, _CHUNK),
                                 jnp.int32),
            jax.ShapeDtypeStruct((_NW, 16), jnp.int32),
        ],
        mesh=_SC_MESH,
        scratch_types=[
            pltpu.VMEM((_BATCH // _CHUNK, _CHUNK), jnp.int32),
            pltpu.VMEM((_WSZ,), jnp.int32),
            pltpu.VMEM((_BATCH // _CHUNK, _CHUNK), jnp.int32),
            pltpu.VMEM((_BATCH // _CHUNK, _CHUNK), jnp.int32),
            pltpu.VMEM((16,), jnp.int32),
        ],
        compiler_params=_SC_PARAMS,
    )(idx2d)


def _write_rows(table_ref, nt_ref, loc_ref, pos_ref, cnt_ref,
                loc_v, pos_v, cnt_v, rsem):
    tab3 = table_ref.reshape(_NGRP, 8, _N_CLASSES)
    nt3 = nt_ref.reshape(_BATCH // 8, 8, _N_CLASSES)
    wid = lax.axis_index("s") * _NC + lax.axis_index("c")
    r_lo = wid * _RPW
    lanes = lax.iota(jnp.int32, 16)

    pltpu.sync_copy(loc_ref.at[wid], loc_v)
    pltpu.sync_copy(pos_ref.at[wid], pos_v)
    pltpu.sync_copy(cnt_ref.at[wid], cnt_v)
    n_live = jnp.max(jnp.where(lanes == 0, cnt_v[pl.ds(0, 16)], 0))

    def row_body(e, _):
        lv = loc_v[e // _CHUNK, pl.ds(((e // 16) % 8) * 16, 16)]
        pv = pos_v[e // _CHUNK, pl.ds(((e // 16) % 8) * 16, 16)]
        sel = lanes == (e % 16)
        loc = jnp.max(jnp.where(sel, lv, 0))
        pos = jnp.max(jnp.where(sel, pv, 0))
        row = jnp.where(loc < _RPW, loc + r_lo, loc - _RPW + _XTRA_ROW0)
        pltpu.async_copy(
            nt3.at[lax.shift_right_logical(pos, 3), pos & 7],
            tab3.at[lax.shift_right_logical(row, 3), row & 7], rsem)
        return 0
    lax.fori_loop(0, n_live, row_body, 0)

    def drain8(e, _):
        pltpu.make_async_copy(nt3.at[0], tab3.at[0], rsem).wait()
        return 0
    lax.fori_loop(0, lax.shift_right_logical(n_live, 3), drain8, 0)

    def drain1(e, _):
        pltpu.make_async_copy(nt3.at[0, 0], tab3.at[0, 0], rsem).wait()
        return 0
    lax.fori_loop(0, n_live & 7, drain1, 0)


def _sc_scatter(table, new_target, loc, pos, cnt):
    def stateful(refs):
        table_ref, nt_ref, loc_ref, pos_ref, cnt_ref = refs

        @pl.core_map(_SC_MESH, compiler_params=_SC_PARAMS,
                     scratch_shapes=[
                         pltpu.VMEM((_BATCH // _CHUNK, _CHUNK), jnp.int32),
                         pltpu.VMEM((_BATCH // _CHUNK, _CHUNK), jnp.int32),
                         pltpu.VMEM((16,), jnp.int32),
                         pltpu.SemaphoreType.DMA,
                     ])
        def _(loc_v, pos_v, cnt_v, rsem):
            _write_rows(table_ref, nt_ref, loc_ref, pos_ref, cnt_ref,
                        loc_v, pos_v, cnt_v, rsem)

    outs = pl.run_state(stateful)((table, new_target, loc, pos, cnt))
    return outs[0]


def kernel(output1, index, confidence):
    gidx2d = (index // 8).reshape(_BATCH // _CHUNK, _CHUNK)
    r82d = (index & 7).reshape(_BATCH // _CHUNK, _CHUNK)
    idx2d = index.reshape(_BATCH // _CHUNK, _CHUNK)
    loc, pos, cnt = _sc_plan(idx2d)
    target128 = _sc_gather(confidence, gidx2d, r82d)
    loss, new_target = _compute_tc(output1, target128)
    new_confidence = _sc_scatter(confidence, new_target, loc, pos, cnt)
    return loss, new_confidence
